# baseline (speedup 1.0000x reference)
import functools

import jax
import jax.numpy as jnp
from jax import lax
from jax.experimental import pallas as pl
from jax.experimental.pallas import tpu as pltpu
from jax.experimental.pallas import tpu_sc as plsc

F32 = jnp.float32
I32 = jnp.int32


def _router_body(x_ref, wr_ref, pos_ref, grow_ref, offs_ref, sb_ref, se_ref,
                 cnt, cntb, offs_s, tril_s, *, TB, E, NB, BM):
    phase = pl.program_id(0)
    b = pl.program_id(1)

    @pl.when((phase == 0) & (b == 0))
    def _init():
        cnt[...] = jnp.zeros_like(cnt)

    xb = x_ref[...]
    w = wr_ref[...]
    logits = jnp.dot(xb, w, preferred_element_type=F32)
    m = jnp.max(logits, axis=1, keepdims=True)
    gate = 1.0 / jnp.sum(jnp.exp(logits - m), axis=1)
    ie = lax.broadcasted_iota(I32, (TB, E), 1)
    eid = jnp.min(jnp.where(logits == m, ie, E), axis=1)
    ie128 = lax.broadcasted_iota(I32, (TB, 128), 1)
    oh128 = (ie128 == eid[:, None]).astype(F32)
    grow_ref[...] = jnp.broadcast_to(gate[:, None], (TB, 128))

    ones_row = jnp.full((1, TB), 1.0, F32)

    @pl.when(phase == 0)
    def _count():
        cnt[0:1, :] = cnt[0:1, :] + jnp.dot(ones_row, oh128,
                                            preferred_element_type=F32)

    @pl.when((phase == 0) & (b == NB - 1))
    def _fin():
        ei = lax.broadcasted_iota(I32, (128, 128), 0)
        ai = lax.broadcasted_iota(I32, (128, 128), 1)
        lt = (ei < ai).astype(F32)
        counts = cnt[0:1, :]
        offs = jnp.dot(counts, lt, precision=lax.Precision.HIGHEST,
                       preferred_element_type=F32)
        offs_s[0:1, :] = offs
        offs_ref[...] = offs.astype(I32)
        starts_i = offs.astype(I32)
        ends_i = (offs + counts).astype(I32)
        fb = starts_i // BM
        nb = jnp.where(counts > 0, (ends_i + BM - 1) // BM - fb, 0)
        nb_f = nb.astype(F32)
        fs = jnp.dot(nb_f, lt, precision=lax.Precision.HIGHEST,
                     preferred_element_type=F32)
        eyem = (ei == ai).astype(F32)
        fs_col = jnp.sum(jnp.broadcast_to(fs, (128, 128)) * eyem,
                         axis=1, keepdims=True)
        nb_col = jnp.sum(jnp.broadcast_to(nb_f, (128, 128)) * eyem,
                         axis=1, keepdims=True)
        fb_col = jnp.sum(jnp.broadcast_to(fb.astype(F32), (128, 128)) * eyem,
                         axis=1, keepdims=True)
        s_f = ai.astype(F32)
        cond = (nb_col > 0.0) & (fs_col <= s_f)
        se_row = jnp.max(jnp.where(cond, ei, -1), axis=0, keepdims=True)
        oh_se = (ei == jnp.broadcast_to(se_row, (128, 128))).astype(F32)
        fb_at = jnp.sum(oh_se * jnp.broadcast_to(fb_col, (128, 128)),
                        axis=0, keepdims=True)
        fs_at = jnp.sum(oh_se * jnp.broadcast_to(fs_col, (128, 128)),
                        axis=0, keepdims=True)
        nb_at = jnp.sum(oh_se * jnp.broadcast_to(nb_col, (128, 128)),
                        axis=0, keepdims=True)
        s_row = lax.broadcasted_iota(I32, (1, 128), 1).astype(F32)
        sb_row = fb_at + (s_row - fs_at)
        sb_row = jnp.minimum(sb_row, fb_at + nb_at - 1.0)
        sb_ref[...] = sb_row.astype(I32)
        se_ref[...] = se_row

    @pl.when((phase == 1) & (b == 0))
    def _initb():
        cntb[...] = jnp.zeros_like(cntb)
        ir = lax.broadcasted_iota(I32, (TB, TB), 0)
        ic = lax.broadcasted_iota(I32, (TB, TB), 1)
        tril_s[...] = (ic < ir).astype(F32)

    @pl.when(phase == 1)
    def _pos():
        rblk = jnp.dot(tril_s[...], oh128, preferred_element_type=F32)
        posf = jnp.sum((rblk + cntb[0:1, :] + offs_s[0:1, :]) * oh128, axis=1)
        cntb[0:1, :] = cntb[0:1, :] + jnp.dot(ones_row, oh128,
                                              preferred_element_type=F32)
        pos_ref[...] = posf.astype(I32).reshape(1, 1, TB)


def _router_call(x, wr, TB, BM):
    T, D = x.shape
    E = wr.shape[1]
    NB = T // TB
    body = functools.partial(_router_body, TB=TB, E=E, NB=NB, BM=BM)
    pos3, grows, offs2, sb2, se2 = pl.pallas_call(
        body,
        grid=(2, NB),
        in_specs=[
            pl.BlockSpec((TB, D), lambda p, s: (s, 0)),
            pl.BlockSpec((D, E), lambda p, s: (0, 0)),
        ],
        out_specs=[
            pl.BlockSpec((1, 1, TB), lambda p, s: (s, 0, 0)),
            pl.BlockSpec((TB, 128), lambda p, s: (s, 0)),
            pl.BlockSpec((1, 128), lambda p, s: (0, 0)),
            pl.BlockSpec((1, 128), lambda p, s: (0, 0)),
            pl.BlockSpec((1, 128), lambda p, s: (0, 0)),
        ],
        out_shape=(
            jax.ShapeDtypeStruct((NB, 1, TB), I32),
            jax.ShapeDtypeStruct((T, 128), F32),
            jax.ShapeDtypeStruct((1, 128), I32),
            jax.ShapeDtypeStruct((1, 128), I32),
            jax.ShapeDtypeStruct((1, 128), I32),
        ),
        scratch_shapes=[pltpu.VMEM((8, 128), F32), pltpu.VMEM((8, 128), F32),
                        pltpu.VMEM((8, 128), F32), pltpu.VMEM((TB, TB), F32)],
    )(x, wr)
    return (pos3.reshape(T), grows, offs2.reshape(128), sb2.reshape(128),
            se2.reshape(128))


def _dispatch_call(x, grows, pos):
    T, D = x.shape
    NW = 32
    CH = T // NW
    mesh = plsc.VectorSubcoreMesh(core_axis_name="c", subcore_axis_name="s")

    @functools.partial(
        pl.kernel,
        mesh=mesh,
        out_type=(
            jax.ShapeDtypeStruct((T, D), F32),
            jax.ShapeDtypeStruct((T, 128), F32),
        ),
        scratch_types=[
            pltpu.VMEM((CH, D), F32),
            pltpu.VMEM((CH, 128), F32),
            pltpu.VMEM((CH,), I32),
            pltpu.SemaphoreType.DMA,
        ],
    )
    def disp(x_hbm, grow_hbm, pos_hbm, xs_hbm, gs_hbm,
             rows_v, grows_v, pos_v, sem):
        wid = lax.axis_index("s") * 2 + lax.axis_index("c")
        base = wid * CH
        pltpu.sync_copy(pos_hbm.at[pl.ds(base, CH)], pos_v)
        pltpu.sync_copy(x_hbm.at[pl.ds(base, CH)], rows_v)
        pltpu.sync_copy(grow_hbm.at[pl.ds(base, CH)], grows_v)
        pltpu.async_copy(rows_v, xs_hbm.at[pos_v], sem).wait()
        pltpu.async_copy(grows_v, gs_hbm.at[pos_v], sem).wait()

    return disp(x, grows, pos)


def _gemm_body(sb_r, se_r, offs_r, x_ref, g_ref, w1_ref, b1_ref, w2_ref,
               b2_ref, out_ref, *, BM, D, F):
    s = pl.program_id(0)
    b = sb_r[s]
    e = se_r[s]
    start = offs_r[e]
    end = offs_r[e + 1]
    x = x_ref[...]
    w1 = w1_ref[...].reshape(D, F)
    h = jax.nn.gelu(jnp.dot(x, w1, preferred_element_type=F32)
                    + b1_ref[...].reshape(1, F))
    w2 = w2_ref[...].reshape(F, D)
    y = jnp.dot(h, w2, preferred_element_type=F32) + b2_ref[...].reshape(1, D)
    y = y * g_ref[...][:, 0:1]
    row = b * BM + lax.broadcasted_iota(I32, (BM, 1), 0)
    mask = (row >= start) & (row < end)
    prev_b = jnp.where(s > 0, sb_r[jnp.maximum(s - 1, 0)], -1)
    old = jnp.where(prev_b == b, out_ref[...], 0.0)
    out_ref[...] = jnp.where(mask, y, old)


def _gemm_call(x_sorted, gate_rows, W1, b1, W2, b2, sb, se, offs, BM, S):
    Ts, D = x_sorted.shape
    E, _, F = W1.shape
    body = functools.partial(_gemm_body, BM=BM, D=D, F=F)
    grid_spec = pltpu.PrefetchScalarGridSpec(
        num_scalar_prefetch=3,
        grid=(S,),
        in_specs=[
            pl.BlockSpec((BM, D), lambda s, sb, se, of: (sb[s], 0)),
            pl.BlockSpec((BM, 128), lambda s, sb, se, of: (sb[s], 0)),
            pl.BlockSpec((1, D, F), lambda s, sb, se, of: (se[s], 0, 0)),
            pl.BlockSpec((1, 1, F), lambda s, sb, se, of: (se[s], 0, 0)),
            pl.BlockSpec((1, F, D), lambda s, sb, se, of: (se[s], 0, 0)),
            pl.BlockSpec((1, 1, D), lambda s, sb, se, of: (se[s], 0, 0)),
        ],
        out_specs=pl.BlockSpec((BM, D), lambda s, sb, se, of: (sb[s], 0)),
    )
    return pl.pallas_call(
        body,
        grid_spec=grid_spec,
        out_shape=jax.ShapeDtypeStruct((Ts, D), F32),
    )(sb, se, offs, x_sorted, gate_rows, W1, b1.reshape(E, 1, F), W2,
      b2.reshape(E, 1, D))


def _combine_call(y_sorted, pos):
    T, D = y_sorted.shape
    NW = 32
    CH = T // NW
    mesh = plsc.VectorSubcoreMesh(core_axis_name="c", subcore_axis_name="s")

    @functools.partial(
        pl.kernel,
        mesh=mesh,
        out_type=jax.ShapeDtypeStruct((T, D), F32),
        scratch_types=[
            pltpu.VMEM((CH,), I32),
            pltpu.VMEM((CH, D), F32),
            pltpu.SemaphoreType.DMA,
        ],
    )
    def comb(ys_hbm, pos_hbm, y_hbm, pos_v, rows_v, sem):
        wid = lax.axis_index("s") * 2 + lax.axis_index("c")
        base = wid * CH
        pltpu.sync_copy(pos_hbm.at[pl.ds(base, CH)], pos_v)
        pltpu.async_copy(ys_hbm.at[pos_v], rows_v, sem).wait()
        pltpu.sync_copy(rows_v, y_hbm.at[pl.ds(base, CH)])

    return comb(y_sorted, pos)


def kernel(hidden_states, W_router, W1, b1, W2, b2):
    T, D = hidden_states.shape
    E = W_router.shape[1]
    F = W1.shape[2]
    TB = 256
    BM = 256
    NB = T // BM
    S = NB + E - 1

    pos, grows, offs, sb, se = _router_call(hidden_states, W_router, TB, BM)
    x_sorted, gate_rows = _dispatch_call(hidden_states, grows, pos)
    y_sorted = _gemm_call(x_sorted, gate_rows, W1, b1, W2, b2, sb, se, offs,
                          BM, S)
    return _combine_call(y_sorted, pos)


# speedup vs baseline: 1.0519x; 1.0519x over previous
import functools

import jax
import jax.numpy as jnp
from jax import lax
from jax.experimental import pallas as pl
from jax.experimental.pallas import tpu as pltpu
from jax.experimental.pallas import tpu_sc as plsc

F32 = jnp.float32
I32 = jnp.int32


def _router_body(x_ref, wr_ref, pos_ref, grow_ref, offs_ref, sb_ref, se_ref,
                 cnt, cntb, offs_s, tril_s, *, TB, E, NB, BM):
    phase = pl.program_id(0)
    b = pl.program_id(1)

    @pl.when((phase == 0) & (b == 0))
    def _init():
        cnt[...] = jnp.zeros_like(cnt)

    xb = x_ref[...]
    w = wr_ref[...]
    logits = jnp.dot(xb, w, preferred_element_type=F32)
    m = jnp.max(logits, axis=1, keepdims=True)
    gate = 1.0 / jnp.sum(jnp.exp(logits - m), axis=1)
    ie = lax.broadcasted_iota(I32, (TB, E), 1)
    eid = jnp.min(jnp.where(logits == m, ie, E), axis=1)
    ie128 = lax.broadcasted_iota(I32, (TB, 128), 1)
    oh128 = (ie128 == eid[:, None]).astype(F32)
    grow_ref[...] = jnp.broadcast_to(gate[:, None], (TB, 128))

    ones_row = jnp.full((1, TB), 1.0, F32)

    @pl.when(phase == 0)
    def _count():
        cnt[0:1, :] = cnt[0:1, :] + jnp.dot(ones_row, oh128,
                                            preferred_element_type=F32)

    @pl.when((phase == 0) & (b == NB - 1))
    def _fin():
        ei = lax.broadcasted_iota(I32, (128, 128), 0)
        ai = lax.broadcasted_iota(I32, (128, 128), 1)
        lt = (ei < ai).astype(F32)
        counts = cnt[0:1, :]
        offs = jnp.dot(counts, lt, precision=lax.Precision.HIGHEST,
                       preferred_element_type=F32)
        offs_s[0:1, :] = offs
        offs_ref[...] = offs.astype(I32)
        starts_i = offs.astype(I32)
        ends_i = (offs + counts).astype(I32)
        fb = starts_i // BM
        nb = jnp.where(counts > 0, (ends_i + BM - 1) // BM - fb, 0)
        nb_f = nb.astype(F32)
        fs = jnp.dot(nb_f, lt, precision=lax.Precision.HIGHEST,
                     preferred_element_type=F32)
        eyem = (ei == ai).astype(F32)
        fs_col = jnp.sum(jnp.broadcast_to(fs, (128, 128)) * eyem,
                         axis=1, keepdims=True)
        nb_col = jnp.sum(jnp.broadcast_to(nb_f, (128, 128)) * eyem,
                         axis=1, keepdims=True)
        fb_col = jnp.sum(jnp.broadcast_to(fb.astype(F32), (128, 128)) * eyem,
                         axis=1, keepdims=True)
        s_f = ai.astype(F32)
        cond = (nb_col > 0.0) & (fs_col <= s_f)
        se_row = jnp.max(jnp.where(cond, ei, -1), axis=0, keepdims=True)
        oh_se = (ei == jnp.broadcast_to(se_row, (128, 128))).astype(F32)
        fb_at = jnp.sum(oh_se * jnp.broadcast_to(fb_col, (128, 128)),
                        axis=0, keepdims=True)
        fs_at = jnp.sum(oh_se * jnp.broadcast_to(fs_col, (128, 128)),
                        axis=0, keepdims=True)
        nb_at = jnp.sum(oh_se * jnp.broadcast_to(nb_col, (128, 128)),
                        axis=0, keepdims=True)
        s_row = lax.broadcasted_iota(I32, (1, 128), 1).astype(F32)
        sb_row = fb_at + (s_row - fs_at)
        sb_row = jnp.minimum(sb_row, fb_at + nb_at - 1.0)
        sb_ref[...] = sb_row.astype(I32)
        se_ref[...] = se_row

    @pl.when((phase == 1) & (b == 0))
    def _initb():
        cntb[...] = jnp.zeros_like(cntb)
        ir = lax.broadcasted_iota(I32, (TB, TB), 0)
        ic = lax.broadcasted_iota(I32, (TB, TB), 1)
        tril_s[...] = (ic < ir).astype(F32)

    @pl.when(phase == 1)
    def _pos():
        rblk = jnp.dot(tril_s[...], oh128, preferred_element_type=F32)
        posf = jnp.sum((rblk + cntb[0:1, :] + offs_s[0:1, :]) * oh128, axis=1)
        cntb[0:1, :] = cntb[0:1, :] + jnp.dot(ones_row, oh128,
                                              preferred_element_type=F32)
        pos_ref[...] = posf.astype(I32).reshape(1, 1, TB)


def _router_call(x, wr, TB, BM):
    T, D = x.shape
    E = wr.shape[1]
    NB = T // TB
    body = functools.partial(_router_body, TB=TB, E=E, NB=NB, BM=BM)
    pos3, grows, offs2, sb2, se2 = pl.pallas_call(
        body,
        grid=(2, NB),
        in_specs=[
            pl.BlockSpec((TB, D), lambda p, s: (s, 0)),
            pl.BlockSpec((D, E), lambda p, s: (0, 0)),
        ],
        out_specs=[
            pl.BlockSpec((1, 1, TB), lambda p, s: (s, 0, 0)),
            pl.BlockSpec((TB, 128), lambda p, s: (s, 0)),
            pl.BlockSpec((1, 128), lambda p, s: (0, 0)),
            pl.BlockSpec((1, 128), lambda p, s: (0, 0)),
            pl.BlockSpec((1, 128), lambda p, s: (0, 0)),
        ],
        out_shape=(
            jax.ShapeDtypeStruct((NB, 1, TB), I32),
            jax.ShapeDtypeStruct((T, 128), F32),
            jax.ShapeDtypeStruct((1, 128), I32),
            jax.ShapeDtypeStruct((1, 128), I32),
            jax.ShapeDtypeStruct((1, 128), I32),
        ),
        scratch_shapes=[pltpu.VMEM((8, 128), F32), pltpu.VMEM((8, 128), F32),
                        pltpu.VMEM((8, 128), F32), pltpu.VMEM((TB, TB), F32)],
    )(x, wr)
    return (pos3.reshape(T), grows, offs2.reshape(128), sb2.reshape(128),
            se2.reshape(128))


def _dispatch_call(x, grows, pos):
    T, D = x.shape
    NW = 32
    CH = T // NW
    mesh = plsc.VectorSubcoreMesh(core_axis_name="c", subcore_axis_name="s")

    @functools.partial(
        pl.kernel,
        mesh=mesh,
        out_type=(
            jax.ShapeDtypeStruct((T, D), F32),
            jax.ShapeDtypeStruct((T, 128), F32),
        ),
        scratch_types=[
            pltpu.VMEM((CH, D), F32),
            pltpu.VMEM((CH, 128), F32),
            pltpu.VMEM((CH,), I32),
            pltpu.SemaphoreType.DMA,
        ],
    )
    def disp(x_hbm, grow_hbm, pos_hbm, xs_hbm, gs_hbm,
             rows_v, grows_v, pos_v, sem):
        wid = lax.axis_index("s") * 2 + lax.axis_index("c")
        base = wid * CH
        pltpu.sync_copy(pos_hbm.at[pl.ds(base, CH)], pos_v)
        pltpu.sync_copy(x_hbm.at[pl.ds(base, CH)], rows_v)
        pltpu.sync_copy(grow_hbm.at[pl.ds(base, CH)], grows_v)
        pltpu.async_copy(rows_v, xs_hbm.at[pos_v], sem).wait()
        pltpu.async_copy(grows_v, gs_hbm.at[pos_v], sem).wait()

    return disp(x, grows, pos)


def _gemm_body(sb_r, se_r, offs_r, x_ref, g_ref, w1_ref, b1_ref, w2_ref,
               b2_ref, out_ref, *, BM, D, F):
    s = pl.program_id(0)
    b = sb_r[s]
    e = se_r[s]
    start = offs_r[e]
    end = offs_r[e + 1]
    x = x_ref[...]
    w1 = w1_ref[...].reshape(D, F)
    h = jax.nn.gelu(jnp.dot(x, w1, preferred_element_type=F32)
                    + b1_ref[...].reshape(1, F))
    w2 = w2_ref[...].reshape(F, D)
    y = jnp.dot(h, w2, preferred_element_type=F32) + b2_ref[...].reshape(1, D)
    y = y * g_ref[...][:, 0:1]
    row = b * BM + lax.broadcasted_iota(I32, (BM, 1), 0)
    mask = (row >= start) & (row < end)
    prev_b = jnp.where(s > 0, sb_r[jnp.maximum(s - 1, 0)], -1)
    old = jnp.where(prev_b == b, out_ref[...], 0.0)
    out_ref[...] = jnp.where(mask, y, old)


def _gemm_call(x_sorted, gate_rows, W1, b1, W2, b2, sb, se, offs, BM, S):
    Ts, D = x_sorted.shape
    E, _, F = W1.shape
    body = functools.partial(_gemm_body, BM=BM, D=D, F=F)
    grid_spec = pltpu.PrefetchScalarGridSpec(
        num_scalar_prefetch=3,
        grid=(S,),
        in_specs=[
            pl.BlockSpec((BM, D), lambda s, sb, se, of: (sb[s], 0)),
            pl.BlockSpec((BM, 128), lambda s, sb, se, of: (sb[s], 0)),
            pl.BlockSpec((1, D, F), lambda s, sb, se, of: (se[s], 0, 0)),
            pl.BlockSpec((1, 1, F), lambda s, sb, se, of: (se[s], 0, 0)),
            pl.BlockSpec((1, F, D), lambda s, sb, se, of: (se[s], 0, 0)),
            pl.BlockSpec((1, 1, D), lambda s, sb, se, of: (se[s], 0, 0)),
        ],
        out_specs=pl.BlockSpec((BM, D), lambda s, sb, se, of: (sb[s], 0)),
    )
    return pl.pallas_call(
        body,
        grid_spec=grid_spec,
        out_shape=jax.ShapeDtypeStruct((Ts, D), F32),
    )(sb, se, offs, x_sorted, gate_rows, W1, b1.reshape(E, 1, F), W2,
      b2.reshape(E, 1, D))


def _combine_call(y_sorted, pos):
    T, D = y_sorted.shape
    NW = 32
    CH = T // NW
    mesh = plsc.VectorSubcoreMesh(core_axis_name="c", subcore_axis_name="s")

    @functools.partial(
        pl.kernel,
        mesh=mesh,
        out_type=jax.ShapeDtypeStruct((T, D), F32),
        scratch_types=[
            pltpu.VMEM((CH,), I32),
            pltpu.VMEM((CH, D), F32),
            pltpu.SemaphoreType.DMA,
        ],
    )
    def comb(ys_hbm, pos_hbm, y_hbm, pos_v, rows_v, sem):
        wid = lax.axis_index("s") * 2 + lax.axis_index("c")
        base = wid * CH
        pltpu.sync_copy(pos_hbm.at[pl.ds(base, CH)], pos_v)
        pltpu.async_copy(ys_hbm.at[pos_v], rows_v, sem).wait()
        pltpu.sync_copy(rows_v, y_hbm.at[pl.ds(base, CH)])

    return comb(y_sorted, pos)


def kernel(hidden_states, W_router, W1, b1, W2, b2):
    T, D = hidden_states.shape
    E = W_router.shape[1]
    F = W1.shape[2]
    TB = 512
    BM = 256
    NB = T // BM
    S = NB + E - 1

    pos, grows, offs, sb, se = _router_call(hidden_states, W_router, TB, BM)
    x_sorted, gate_rows = _dispatch_call(hidden_states, grows, pos)
    y_sorted = _gemm_call(x_sorted, gate_rows, W1, b1, W2, b2, sb, se, offs,
                          BM, S)
    return _combine_call(y_sorted, pos)


# speedup vs baseline: 1.0576x; 1.0054x over previous
import functools

import jax
import jax.numpy as jnp
from jax import lax
from jax.experimental import pallas as pl
from jax.experimental.pallas import tpu as pltpu
from jax.experimental.pallas import tpu_sc as plsc

F32 = jnp.float32
I32 = jnp.int32


def _router_body(x_ref, wr_ref, pos_ref, grow_ref, offs_ref, sb_ref, se_ref,
                 cnt, cntb, offs_s, tril_s, *, TB, E, NB, BM):
    phase = pl.program_id(0)
    b = pl.program_id(1)

    @pl.when((phase == 0) & (b == 0))
    def _init():
        cnt[...] = jnp.zeros_like(cnt)

    xb = x_ref[...]
    w = wr_ref[...]
    logits = jnp.dot(xb, w, preferred_element_type=F32)
    m = jnp.max(logits, axis=1, keepdims=True)
    gate = 1.0 / jnp.sum(jnp.exp(logits - m), axis=1)
    ie = lax.broadcasted_iota(I32, (TB, E), 1)
    eid = jnp.min(jnp.where(logits == m, ie, E), axis=1)
    ie128 = lax.broadcasted_iota(I32, (TB, 128), 1)
    oh128 = (ie128 == eid[:, None]).astype(F32)
    grow_ref[...] = jnp.broadcast_to(gate[:, None], (TB, 128))

    ones_row = jnp.full((1, TB), 1.0, F32)

    @pl.when(phase == 0)
    def _count():
        cnt[0:1, :] = cnt[0:1, :] + jnp.dot(ones_row, oh128,
                                            preferred_element_type=F32)

    @pl.when((phase == 0) & (b == NB - 1))
    def _fin():
        ei = lax.broadcasted_iota(I32, (128, 128), 0)
        ai = lax.broadcasted_iota(I32, (128, 128), 1)
        lt = (ei < ai).astype(F32)
        counts = cnt[0:1, :]
        offs = jnp.dot(counts, lt, precision=lax.Precision.HIGHEST,
                       preferred_element_type=F32)
        offs_s[0:1, :] = offs
        offs_ref[...] = offs.astype(I32)
        starts_i = offs.astype(I32)
        ends_i = (offs + counts).astype(I32)
        fb = starts_i // BM
        nb = jnp.where(counts > 0, (ends_i + BM - 1) // BM - fb, 0)
        nb_f = nb.astype(F32)
        fs = jnp.dot(nb_f, lt, precision=lax.Precision.HIGHEST,
                     preferred_element_type=F32)
        eyem = (ei == ai).astype(F32)
        fs_col = jnp.sum(jnp.broadcast_to(fs, (128, 128)) * eyem,
                         axis=1, keepdims=True)
        nb_col = jnp.sum(jnp.broadcast_to(nb_f, (128, 128)) * eyem,
                         axis=1, keepdims=True)
        fb_col = jnp.sum(jnp.broadcast_to(fb.astype(F32), (128, 128)) * eyem,
                         axis=1, keepdims=True)
        s_f = ai.astype(F32)
        cond = (nb_col > 0.0) & (fs_col <= s_f)
        se_row = jnp.max(jnp.where(cond, ei, -1), axis=0, keepdims=True)
        oh_se = (ei == jnp.broadcast_to(se_row, (128, 128))).astype(F32)
        fb_at = jnp.sum(oh_se * jnp.broadcast_to(fb_col, (128, 128)),
                        axis=0, keepdims=True)
        fs_at = jnp.sum(oh_se * jnp.broadcast_to(fs_col, (128, 128)),
                        axis=0, keepdims=True)
        nb_at = jnp.sum(oh_se * jnp.broadcast_to(nb_col, (128, 128)),
                        axis=0, keepdims=True)
        s_row = lax.broadcasted_iota(I32, (1, 128), 1).astype(F32)
        sb_row = fb_at + (s_row - fs_at)
        sb_row = jnp.minimum(sb_row, fb_at + nb_at - 1.0)
        sb_ref[...] = sb_row.astype(I32)
        se_ref[...] = se_row

    @pl.when((phase == 1) & (b == 0))
    def _initb():
        cntb[...] = jnp.zeros_like(cntb)
        ir = lax.broadcasted_iota(I32, (TB, TB), 0)
        ic = lax.broadcasted_iota(I32, (TB, TB), 1)
        tril_s[...] = (ic < ir).astype(F32)

    @pl.when(phase == 1)
    def _pos():
        rblk = jnp.dot(tril_s[...], oh128, preferred_element_type=F32)
        posf = jnp.sum((rblk + cntb[0:1, :] + offs_s[0:1, :]) * oh128, axis=1)
        cntb[0:1, :] = cntb[0:1, :] + jnp.dot(ones_row, oh128,
                                              preferred_element_type=F32)
        pos_ref[...] = posf.astype(I32).reshape(1, 1, TB)


def _router_call(x, wr, TB, BM):
    T, D = x.shape
    E = wr.shape[1]
    NB = T // TB
    body = functools.partial(_router_body, TB=TB, E=E, NB=NB, BM=BM)
    pos3, grows, offs2, sb2, se2 = pl.pallas_call(
        body,
        grid=(2, NB),
        in_specs=[
            pl.BlockSpec((TB, D), lambda p, s: (s, 0)),
            pl.BlockSpec((D, E), lambda p, s: (0, 0)),
        ],
        out_specs=[
            pl.BlockSpec((1, 1, TB), lambda p, s: (s, 0, 0)),
            pl.BlockSpec((TB, 128), lambda p, s: (s, 0)),
            pl.BlockSpec((1, 128), lambda p, s: (0, 0)),
            pl.BlockSpec((1, 128), lambda p, s: (0, 0)),
            pl.BlockSpec((1, 128), lambda p, s: (0, 0)),
        ],
        out_shape=(
            jax.ShapeDtypeStruct((NB, 1, TB), I32),
            jax.ShapeDtypeStruct((T, 128), F32),
            jax.ShapeDtypeStruct((1, 128), I32),
            jax.ShapeDtypeStruct((1, 128), I32),
            jax.ShapeDtypeStruct((1, 128), I32),
        ),
        scratch_shapes=[pltpu.VMEM((8, 128), F32), pltpu.VMEM((8, 128), F32),
                        pltpu.VMEM((8, 128), F32), pltpu.VMEM((TB, TB), F32)],
    )(x, wr)
    return (pos3.reshape(T), grows, offs2.reshape(128), sb2.reshape(128),
            se2.reshape(128))


def _dispatch_call(x, grows, pos):
    T, D = x.shape
    NW = 32
    CH = T // NW
    mesh = plsc.VectorSubcoreMesh(core_axis_name="c", subcore_axis_name="s")

    @functools.partial(
        pl.kernel,
        mesh=mesh,
        out_type=(
            jax.ShapeDtypeStruct((T, D), F32),
            jax.ShapeDtypeStruct((T, 128), F32),
        ),
        scratch_types=[
            pltpu.VMEM((CH, D), F32),
            pltpu.VMEM((CH, 128), F32),
            pltpu.VMEM((CH,), I32),
            pltpu.SemaphoreType.DMA,
            pltpu.SemaphoreType.DMA,
        ],
    )
    def disp(x_hbm, grow_hbm, pos_hbm, xs_hbm, gs_hbm,
             rows_v, grows_v, pos_v, sem, sem2):
        wid = lax.axis_index("s") * 2 + lax.axis_index("c")
        base = wid * CH
        cpx = pltpu.async_copy(x_hbm.at[pl.ds(base, CH)], rows_v, sem)
        cpg = pltpu.async_copy(grow_hbm.at[pl.ds(base, CH)], grows_v, sem2)
        pltpu.sync_copy(pos_hbm.at[pl.ds(base, CH)], pos_v)
        cpx.wait()
        cpg.wait()
        s1 = pltpu.async_copy(rows_v, xs_hbm.at[pos_v], sem)
        s2 = pltpu.async_copy(grows_v, gs_hbm.at[pos_v], sem2)
        s1.wait()
        s2.wait()

    return disp(x, grows, pos)


def _gemm_body(sb_r, se_r, offs_r, x_ref, g_ref, w1_ref, b1_ref, w2_ref,
               b2_ref, out_ref, *, BM, D, F):
    s = pl.program_id(0)
    b = sb_r[s]
    e = se_r[s]
    start = offs_r[e]
    end = offs_r[e + 1]
    x = x_ref[...]
    w1 = w1_ref[...].reshape(D, F)
    h = jax.nn.gelu(jnp.dot(x, w1, preferred_element_type=F32)
                    + b1_ref[...].reshape(1, F))
    w2 = w2_ref[...].reshape(F, D)
    y = jnp.dot(h, w2, preferred_element_type=F32) + b2_ref[...].reshape(1, D)
    y = y * g_ref[...][:, 0:1]
    row = b * BM + lax.broadcasted_iota(I32, (BM, 1), 0)
    mask = (row >= start) & (row < end)
    prev_b = jnp.where(s > 0, sb_r[jnp.maximum(s - 1, 0)], -1)
    old = jnp.where(prev_b == b, out_ref[...], 0.0)
    out_ref[...] = jnp.where(mask, y, old)


def _gemm_call(x_sorted, gate_rows, W1, b1, W2, b2, sb, se, offs, BM, S):
    Ts, D = x_sorted.shape
    E, _, F = W1.shape
    body = functools.partial(_gemm_body, BM=BM, D=D, F=F)
    grid_spec = pltpu.PrefetchScalarGridSpec(
        num_scalar_prefetch=3,
        grid=(S,),
        in_specs=[
            pl.BlockSpec((BM, D), lambda s, sb, se, of: (sb[s], 0)),
            pl.BlockSpec((BM, 128), lambda s, sb, se, of: (sb[s], 0)),
            pl.BlockSpec((1, D, F), lambda s, sb, se, of: (se[s], 0, 0)),
            pl.BlockSpec((1, 1, F), lambda s, sb, se, of: (se[s], 0, 0)),
            pl.BlockSpec((1, F, D), lambda s, sb, se, of: (se[s], 0, 0)),
            pl.BlockSpec((1, 1, D), lambda s, sb, se, of: (se[s], 0, 0)),
        ],
        out_specs=pl.BlockSpec((BM, D), lambda s, sb, se, of: (sb[s], 0)),
    )
    return pl.pallas_call(
        body,
        grid_spec=grid_spec,
        out_shape=jax.ShapeDtypeStruct((Ts, D), F32),
    )(sb, se, offs, x_sorted, gate_rows, W1, b1.reshape(E, 1, F), W2,
      b2.reshape(E, 1, D))


def _combine_call(y_sorted, pos):
    T, D = y_sorted.shape
    NW = 32
    CH = T // NW
    mesh = plsc.VectorSubcoreMesh(core_axis_name="c", subcore_axis_name="s")

    @functools.partial(
        pl.kernel,
        mesh=mesh,
        out_type=jax.ShapeDtypeStruct((T, D), F32),
        scratch_types=[
            pltpu.VMEM((CH,), I32),
            pltpu.VMEM((CH, D), F32),
            pltpu.SemaphoreType.DMA,
        ],
    )
    def comb(ys_hbm, pos_hbm, y_hbm, pos_v, rows_v, sem):
        wid = lax.axis_index("s") * 2 + lax.axis_index("c")
        base = wid * CH
        pltpu.sync_copy(pos_hbm.at[pl.ds(base, CH)], pos_v)
        pltpu.async_copy(ys_hbm.at[pos_v], rows_v, sem).wait()
        pltpu.sync_copy(rows_v, y_hbm.at[pl.ds(base, CH)])

    return comb(y_sorted, pos)


def kernel(hidden_states, W_router, W1, b1, W2, b2):
    T, D = hidden_states.shape
    E = W_router.shape[1]
    F = W1.shape[2]
    TB = 512
    BM = 256
    NB = T // BM
    S = NB + E - 1

    pos, grows, offs, sb, se = _router_call(hidden_states, W_router, TB, BM)
    x_sorted, gate_rows = _dispatch_call(hidden_states, grows, pos)
    y_sorted = _gemm_call(x_sorted, gate_rows, W1, b1, W2, b2, sb, se, offs,
                          BM, S)
    return _combine_call(y_sorted, pos)
